# baseline (speedup 1.0000x reference)
import jax
import jax.numpy as jnp
from jax import lax
from jax.experimental import pallas as pl
from jax.experimental.pallas import tpu as pltpu
from jax.experimental.pallas import tpu_sc as plsc

_BATCH = 16384
_FIELDS = 26
_CHUNK_OUT = 32
_N_CHUNKS = 4
_OUT_DIM = _N_CHUNKS * _CHUNK_OUT
_B_FLAT = _BATCH * _FIELDS
_L = 128
_NW = 32
_ROWS_PER_W = _B_FLAT // (_NW * _L)
_TROWS = 1000000 // _N_CHUNKS

_RP_PER_W = 7816
_RP_CHUNK = 64
_RP_ITERS = -(-_RP_PER_W // _RP_CHUNK)

_mesh = plsc.VectorSubcoreMesh(core_axis_name="core", subcore_axis_name="subcore")


def _repack(table_0, table_1, table_2, table_3):
    ot = jax.ShapeDtypeStruct((_TROWS, _OUT_DIM), jnp.float32)

    @pl.kernel(
        out_type=(ot, ot, ot, ot),
        mesh=_mesh,
        scratch_types=[
            pltpu.VMEM((_RP_CHUNK * 4, _CHUNK_OUT), jnp.float32),
            pltpu.VMEM((_RP_CHUNK, _OUT_DIM), jnp.float32),
        ],
    )
    def k(t0, t1, t2, t3, o0, o1, o2, o3, ibuf, obuf):
        wid = lax.axis_index("subcore") * 2 + lax.axis_index("core")
        base = wid * _RP_PER_W
        lim = jnp.minimum(base + _RP_PER_W, _TROWS)

        for t_in, t_out in ((t0, o0), (t1, o1), (t2, o2), (t3, o3)):

            @pl.loop(0, _RP_ITERS)
            def _(c):
                start = jnp.minimum(base + c * _RP_CHUNK, lim - _RP_CHUNK)
                pltpu.sync_copy(t_in.at[pl.ds(start * 4, _RP_CHUNK * 4)], ibuf)

                @pl.loop(0, _RP_CHUNK)
                def _(j):
                    for q in range(4):
                        for h in range(_CHUNK_OUT // 16):
                            obuf[j, pl.ds(q * _CHUNK_OUT + h * 16, 16)] = ibuf[
                                j * 4 + q, pl.ds(h * 16, 16)
                            ]

                pltpu.sync_copy(obuf, t_out.at[pl.ds(start, _RP_CHUNK)])

    return k(table_0, table_1, table_2, table_3)


@jax.jit
def kernel(indices, table_0, table_1, table_2, table_3):
    idx = indices.reshape(_B_FLAT // _L, _L).astype(jnp.int32)
    tv = _repack(table_0, table_1, table_2, table_3)

    @pl.kernel(
        out_type=jax.ShapeDtypeStruct((_B_FLAT, _OUT_DIM), jnp.float32),
        mesh=_mesh,
        scratch_types=[
            pltpu.VMEM((2, _L), jnp.int32),
            pltpu.VMEM((2, _L), jnp.int32),
            pltpu.VMEM((_L, _OUT_DIM), jnp.float32),
            pltpu.VMEM((_L, _OUT_DIM), jnp.float32),
            pltpu.VMEM((2, _L, _OUT_DIM), jnp.float32),
            pltpu.SemaphoreType.DMA,
            pltpu.SemaphoreType.DMA,
            pltpu.SemaphoreType.DMA,
            pltpu.SemaphoreType.DMA,
            pltpu.SemaphoreType.DMA,
        ],
    )
    def k(idx_hbm, t0_hbm, t1_hbm, t2_hbm, t3_hbm, o_hbm,
          idx_v, q_v, gb0, gb1, obuf, sg0, sg1, si, so0, so1):
        tables = (t0_hbm, t1_hbm, t2_hbm, t3_hbm)
        sos = (so0, so1)
        wid = lax.axis_index("subcore") * 2 + lax.axis_index("core")
        row0 = wid * _ROWS_PER_W

        def compute_q(p):
            for v in range(_L // 16):
                q_v[p, pl.ds(v * 16, 16)] = idx_v[p, pl.ds(v * 16, 16)] >> 2

        def assemble(gb, p, t):
            @pl.loop(0, _L // 16)
            def _(g):
                offs = (idx_v[p, pl.ds(g * 16, 16)] & 3) * _CHUNK_OUT
                for l in range(16):
                    off = offs[l]
                    for h in range(_CHUNK_OUT // 16):
                        obuf[p, g * 16 + l, pl.ds(t * _CHUNK_OUT + h * 16, 16)] = (
                            gb[g * 16 + l, pl.ds(off + h * 16, 16)]
                        )

        def chunk_body(c, p):
            pn = 1 - p
            irow = row0 + c

            @pl.when(c < _ROWS_PER_W - 1)
            def _():
                pltpu.async_copy(idx_hbm.at[irow + 1], idx_v.at[pn], si)

            @pl.when(c >= 2)
            def _():
                pltpu.make_async_copy(
                    obuf.at[p], o_hbm.at[pl.ds((irow - 2) * _L, _L)], sos[p]
                ).wait()

            pltpu.async_copy(tables[1].at[q_v.at[p]], gb1, sg1)
            pltpu.make_async_copy(tables[0].at[q_v.at[p]], gb0, sg0).wait()
            assemble(gb0, p, 0)

            pltpu.async_copy(tables[2].at[q_v.at[p]], gb0, sg0)
            pltpu.make_async_copy(tables[1].at[q_v.at[p]], gb1, sg1).wait()
            assemble(gb1, p, 1)

            pltpu.async_copy(tables[3].at[q_v.at[p]], gb1, sg1)
            pltpu.make_async_copy(tables[2].at[q_v.at[p]], gb0, sg0).wait()
            assemble(gb0, p, 2)

            @pl.when(c < _ROWS_PER_W - 1)
            def _():
                pltpu.make_async_copy(idx_hbm.at[irow + 1], idx_v.at[pn], si).wait()
                compute_q(pn)
                pltpu.async_copy(tables[0].at[q_v.at[pn]], gb0, sg0)

            pltpu.make_async_copy(tables[3].at[q_v.at[p]], gb1, sg1).wait()
            assemble(gb1, p, 3)

            pltpu.async_copy(obuf.at[p], o_hbm.at[pl.ds(irow * _L, _L)], sos[p])

        pltpu.sync_copy(idx_hbm.at[row0], idx_v.at[0])
        compute_q(0)
        pltpu.async_copy(tables[0].at[q_v.at[0]], gb0, sg0)

        @pl.loop(0, _ROWS_PER_W // 2)
        def _(cc):
            chunk_body(cc * 2, 0)
            chunk_body(cc * 2 + 1, 1)

        last = row0 + _ROWS_PER_W - 1
        pltpu.make_async_copy(
            obuf.at[0], o_hbm.at[pl.ds((last - 1) * _L, _L)], so0
        ).wait()
        pltpu.make_async_copy(
            obuf.at[1], o_hbm.at[pl.ds(last * _L, _L)], so1
        ).wait()

    out = k(idx, *tv)
    return out.reshape(_BATCH, _FIELDS, _OUT_DIM)


# speedup vs baseline: 1.5630x; 1.5630x over previous
import jax
import jax.numpy as jnp
from jax import lax
from jax.experimental import pallas as pl
from jax.experimental.pallas import tpu as pltpu
from jax.experimental.pallas import tpu_sc as plsc

_BATCH = 16384
_FIELDS = 26
_CHUNK_OUT = 32
_N_CHUNKS = 4
_OUT_DIM = _N_CHUNKS * _CHUNK_OUT
_B_FLAT = _BATCH * _FIELDS
_L = 128
_NW = 32
_ROWS_PER_W = _B_FLAT // (_NW * _L)
_TROWS = 1000000
_BR = 2000

_mesh = plsc.VectorSubcoreMesh(core_axis_name="core", subcore_axis_name="subcore")


def _concat_tables(t0, t1, t2, t3):
    def body(r0, r1, r2, r3, o_ref):
        for t, r in enumerate((r0, r1, r2, r3)):
            o_ref[:, t * _CHUNK_OUT:(t + 1) * _CHUNK_OUT] = r[...]

    return pl.pallas_call(
        body,
        grid=(_TROWS // _BR,),
        in_specs=[pl.BlockSpec((_BR, _CHUNK_OUT), lambda i: (i, 0))] * 4,
        out_specs=pl.BlockSpec((_BR, _OUT_DIM), lambda i: (i, 0)),
        out_shape=jax.ShapeDtypeStruct((_TROWS, _OUT_DIM), jnp.float32),
    )(t0, t1, t2, t3)


@jax.jit
def kernel(indices, table_0, table_1, table_2, table_3):
    idx = indices.reshape(_B_FLAT // _L, _L).astype(jnp.int32)
    tcat = _concat_tables(table_0, table_1, table_2, table_3)

    @pl.kernel(
        out_type=jax.ShapeDtypeStruct((_B_FLAT, _OUT_DIM), jnp.float32),
        mesh=_mesh,
        scratch_types=[
            pltpu.VMEM((2, _L), jnp.int32),
            pltpu.VMEM((_L, _OUT_DIM), jnp.float32),
            pltpu.VMEM((_L, _OUT_DIM), jnp.float32),
            pltpu.SemaphoreType.DMA,
            pltpu.SemaphoreType.DMA,
            pltpu.SemaphoreType.DMA,
            pltpu.SemaphoreType.DMA,
            pltpu.SemaphoreType.DMA,
        ],
    )
    def k(idx_hbm, t_hbm, o_hbm, idx_v, gb0, gb1, sg0, sg1, si, so0, so1):
        gbs = (gb0, gb1)
        sgs = (sg0, sg1)
        sos = (so0, so1)
        wid = lax.axis_index("subcore") * 2 + lax.axis_index("core")
        row0 = wid * _ROWS_PER_W

        def chunk_body(c, p):
            pn = 1 - p
            irow = row0 + c

            @pl.when(c < _ROWS_PER_W - 1)
            def _():
                pltpu.async_copy(idx_hbm.at[irow + 1], idx_v.at[pn], si)

            pltpu.make_async_copy(t_hbm.at[idx_v.at[p]], gbs[p], sgs[p]).wait()
            pltpu.async_copy(gbs[p], o_hbm.at[pl.ds(irow * _L, _L)], sos[p])

            @pl.when(c < _ROWS_PER_W - 1)
            def _():
                pltpu.make_async_copy(idx_hbm.at[irow + 1], idx_v.at[pn], si).wait()

                @pl.when(c >= 1)
                def _():
                    pltpu.make_async_copy(
                        gbs[pn], o_hbm.at[pl.ds((irow - 1) * _L, _L)], sos[pn]
                    ).wait()

                pltpu.async_copy(t_hbm.at[idx_v.at[pn]], gbs[pn], sgs[pn])

        pltpu.sync_copy(idx_hbm.at[row0], idx_v.at[0])
        pltpu.async_copy(t_hbm.at[idx_v.at[0]], gb0, sg0)

        @pl.loop(0, _ROWS_PER_W // 2)
        def _(cc):
            chunk_body(cc * 2, 0)
            chunk_body(cc * 2 + 1, 1)

        last = row0 + _ROWS_PER_W - 1
        pltpu.make_async_copy(
            gb0, o_hbm.at[pl.ds((last - 1) * _L, _L)], so0
        ).wait()
        pltpu.make_async_copy(
            gb1, o_hbm.at[pl.ds(last * _L, _L)], so1
        ).wait()

    out = k(idx, tcat)
    return out.reshape(_BATCH, _FIELDS, _OUT_DIM)


# speedup vs baseline: 1.8718x; 1.1975x over previous
import jax
import jax.numpy as jnp
from jax import lax
from jax.experimental import pallas as pl
from jax.experimental.pallas import tpu as pltpu
from jax.experimental.pallas import tpu_sc as plsc

_BATCH = 16384
_FIELDS = 26
_CHUNK_OUT = 32
_N_CHUNKS = 4
_OUT_DIM = _N_CHUNKS * _CHUNK_OUT
_B_FLAT = _BATCH * _FIELDS
_L = 128
_NW = 32
_ROWS_PER_W = _B_FLAT // (_NW * _L)

_mesh = plsc.VectorSubcoreMesh(core_axis_name="core", subcore_axis_name="subcore")


@jax.jit
def kernel(indices, table_0, table_1, table_2, table_3):
    idx = indices.reshape(_B_FLAT // _L, _L).astype(jnp.int32)
    pad = jnp.zeros((1000000, 2 * _CHUNK_OUT), jnp.float32)
    t01 = jnp.concatenate([table_0, table_1, pad], axis=1)
    t23 = jnp.concatenate([table_2, table_3, pad], axis=1)

    @pl.kernel(
        out_type=jax.ShapeDtypeStruct((_B_FLAT, _OUT_DIM), jnp.float32),
        mesh=_mesh,
        scratch_types=[
            pltpu.VMEM((2, _L), jnp.int32),
            pltpu.VMEM((_L, _OUT_DIM), jnp.float32),
            pltpu.VMEM((_L, _OUT_DIM), jnp.float32),
            pltpu.VMEM((2, _L, _OUT_DIM), jnp.float32),
            pltpu.SemaphoreType.DMA,
            pltpu.SemaphoreType.DMA,
            pltpu.SemaphoreType.DMA,
            pltpu.SemaphoreType.DMA,
            pltpu.SemaphoreType.DMA,
        ],
    )
    def k(idx_hbm, t01_hbm, t23_hbm, o_hbm,
          idx_v, gb0, gb1, obuf, sg0, sg1, si, so0, so1):
        tables = (t01_hbm, t23_hbm)
        sos = (so0, so1)
        wid = lax.axis_index("subcore") * 2 + lax.axis_index("core")
        row0 = wid * _ROWS_PER_W

        def assemble(gb, p, t):
            @pl.loop(0, _L)
            def _(r):
                for h in range(2 * _CHUNK_OUT // 16):
                    obuf[p, r, pl.ds(t * 2 * _CHUNK_OUT + h * 16, 16)] = gb[
                        r, pl.ds(h * 16, 16)
                    ]

        def chunk_body(c, p):
            pn = 1 - p
            irow = row0 + c

            @pl.when(c < _ROWS_PER_W - 1)
            def _():
                pltpu.async_copy(idx_hbm.at[irow + 1], idx_v.at[pn], si)

            @pl.when(c >= 2)
            def _():
                pltpu.make_async_copy(
                    obuf.at[p], o_hbm.at[pl.ds((irow - 2) * _L, _L)], sos[p]
                ).wait()

            pltpu.async_copy(tables[1].at[idx_v.at[p]], gb1, sg1)
            pltpu.make_async_copy(tables[0].at[idx_v.at[p]], gb0, sg0).wait()
            assemble(gb0, p, 0)

            @pl.when(c < _ROWS_PER_W - 1)
            def _():
                pltpu.make_async_copy(idx_hbm.at[irow + 1], idx_v.at[pn], si).wait()
                pltpu.async_copy(tables[0].at[idx_v.at[pn]], gb0, sg0)

            pltpu.make_async_copy(tables[1].at[idx_v.at[p]], gb1, sg1).wait()
            assemble(gb1, p, 1)

            pltpu.async_copy(obuf.at[p], o_hbm.at[pl.ds(irow * _L, _L)], sos[p])

        pltpu.sync_copy(idx_hbm.at[row0], idx_v.at[0])
        pltpu.async_copy(t01_hbm.at[idx_v.at[0]], gb0, sg0)

        @pl.loop(0, _ROWS_PER_W // 2)
        def _(cc):
            chunk_body(cc * 2, 0)
            chunk_body(cc * 2 + 1, 1)

        last = row0 + _ROWS_PER_W - 1
        pltpu.make_async_copy(
            obuf.at[0], o_hbm.at[pl.ds((last - 1) * _L, _L)], so0
        ).wait()
        pltpu.make_async_copy(
            obuf.at[1], o_hbm.at[pl.ds(last * _L, _L)], so1
        ).wait()

    out = k(idx, t01, t23)
    return out.reshape(_BATCH, _FIELDS, _OUT_DIM)


# speedup vs baseline: 2.0770x; 1.1097x over previous
import jax
import jax.numpy as jnp
from jax import lax
from jax.experimental import pallas as pl
from jax.experimental.pallas import tpu as pltpu
from jax.experimental.pallas import tpu_sc as plsc

_BATCH = 16384
_FIELDS = 26
_CHUNK_OUT = 32
_N_CHUNKS = 4
_OUT_DIM = _N_CHUNKS * _CHUNK_OUT
_B_FLAT = _BATCH * _FIELDS
_L = 128
_NW = 32
_ROWS_PER_W = _B_FLAT // (_NW * _L)
_TROWS = 1000000
_BR = 2048

_mesh = plsc.VectorSubcoreMesh(core_axis_name="core", subcore_axis_name="subcore")


def _concat_tables(t0, t1, t2, t3):
    return jnp.concatenate([t0, t1, t2, t3], axis=1)


@jax.jit
def kernel(indices, table_0, table_1, table_2, table_3):
    idx = indices.reshape(_B_FLAT // _L, _L).astype(jnp.int32)
    tcat = _concat_tables(table_0, table_1, table_2, table_3)

    @pl.kernel(
        out_type=jax.ShapeDtypeStruct((_B_FLAT, _OUT_DIM), jnp.float32),
        mesh=_mesh,
        scratch_types=[
            pltpu.VMEM((2, _L), jnp.int32),
            pltpu.VMEM((_L, _OUT_DIM), jnp.float32),
            pltpu.VMEM((_L, _OUT_DIM), jnp.float32),
            pltpu.SemaphoreType.DMA,
            pltpu.SemaphoreType.DMA,
            pltpu.SemaphoreType.DMA,
            pltpu.SemaphoreType.DMA,
            pltpu.SemaphoreType.DMA,
        ],
    )
    def k(idx_hbm, t_hbm, o_hbm, idx_v, gb0, gb1, sg0, sg1, si, so0, so1):
        gbs = (gb0, gb1)
        sgs = (sg0, sg1)
        sos = (so0, so1)
        wid = lax.axis_index("subcore") * 2 + lax.axis_index("core")
        row0 = wid * _ROWS_PER_W

        def chunk_body(c, p):
            pn = 1 - p
            irow = row0 + c

            @pl.when(c < _ROWS_PER_W - 1)
            def _():
                pltpu.async_copy(idx_hbm.at[irow + 1], idx_v.at[pn], si)

            pltpu.make_async_copy(t_hbm.at[idx_v.at[p]], gbs[p], sgs[p]).wait()
            pltpu.async_copy(gbs[p], o_hbm.at[pl.ds(irow * _L, _L)], sos[p])

            @pl.when(c < _ROWS_PER_W - 1)
            def _():
                pltpu.make_async_copy(idx_hbm.at[irow + 1], idx_v.at[pn], si).wait()

                @pl.when(c >= 1)
                def _():
                    pltpu.make_async_copy(
                        gbs[pn], o_hbm.at[pl.ds((irow - 1) * _L, _L)], sos[pn]
                    ).wait()

                pltpu.async_copy(t_hbm.at[idx_v.at[pn]], gbs[pn], sgs[pn])

        pltpu.sync_copy(idx_hbm.at[row0], idx_v.at[0])
        pltpu.async_copy(t_hbm.at[idx_v.at[0]], gb0, sg0)

        @pl.loop(0, _ROWS_PER_W // 2)
        def _(cc):
            chunk_body(cc * 2, 0)
            chunk_body(cc * 2 + 1, 1)

        last = row0 + _ROWS_PER_W - 1
        pltpu.make_async_copy(
            gb0, o_hbm.at[pl.ds((last - 1) * _L, _L)], so0
        ).wait()
        pltpu.make_async_copy(
            gb1, o_hbm.at[pl.ds(last * _L, _L)], so1
        ).wait()

    out = k(idx, tcat)
    return out.reshape(_BATCH, _FIELDS, _OUT_DIM)
